# baseline (device time: 84415 ns/iter reference)
import jax
import jax.numpy as jnp
from jax import lax
from jax.experimental import pallas as pl
from jax.experimental.pallas import tpu as pltpu

N_DEV = 4
B, SQ, D = 4, 256, 1024
DH2 = D // 2
HQ, DH = 8, 128
SCALE = 0.08838834764831843

P1 = [1, 0, 3, 2]
P2 = [3, 2, 1, 0]
H0 = [0, 2, 2, 0]
KB = [0, 2, 3, 1]
BP1 = [3, 2, 1, 0]
BP2 = [1, 0, 3, 2]
BH0 = [0, 0, 2, 2]
BKB = [0, 1, 2, 3]


def kernel(x, Wq, Wo, Wk, Wv):
    def body(x_ref, wq_ref, wo_ref, wk_ref, wv_ref, out_ref,
             pa_ref, pb_ref, r1a_ref, r1b_ref, aa_ref, ab_ref,
             ga_ref, gb_ref, send_sems, recv_sems):
        my = lax.axis_index("i")
        left = (my - 1) % N_DEV
        right = (my + 1) % N_DEV

        barrier_sem = pltpu.get_barrier_semaphore()
        for nbr in (left, right):
            pl.semaphore_signal(
                barrier_sem, inc=1,
                device_id=(nbr,), device_id_type=pl.DeviceIdType.MESH,
            )
        pl.semaphore_wait(barrier_sem, 2)

        wq = wq_ref[...].astype(jnp.bfloat16)
        wk = wk_ref[...].astype(jnp.bfloat16)
        wv = wv_ref[...].astype(jnp.bfloat16)
        wo = wo_ref[...].astype(jnp.bfloat16)

        def compute_batch(b):
            xb = x_ref[b].astype(jnp.bfloat16)
            qb = jnp.dot(xb, wq, preferred_element_type=jnp.float32)
            kb_ = jnp.dot(xb, wk, preferred_element_type=jnp.float32)
            vb = jnp.dot(xb, wv, preferred_element_type=jnp.float32)
            qb = qb.astype(jnp.bfloat16)
            kb_ = kb_.astype(jnp.bfloat16)
            vb = vb.astype(jnp.bfloat16)
            heads = []
            for h in range(HQ):
                sl = slice(h * DH, (h + 1) * DH)
                q, k, v = qb[:, sl], kb_[:, sl], vb[:, sl]
                s = lax.dot_general(
                    q, k, (((1,), (1,)), ((), ())),
                    preferred_element_type=jnp.float32,
                ) * SCALE
                m = jnp.max(s, axis=1, keepdims=True)
                p = jnp.exp(s - m)
                l = jnp.sum(p, axis=1, keepdims=True)
                o = jnp.dot(p.astype(jnp.bfloat16), v,
                            preferred_element_type=jnp.float32) / l
                heads.append(o.astype(jnp.bfloat16))
            ob = jnp.concatenate(heads, axis=1)
            partial = jnp.dot(
                ob, wo, preferred_element_type=jnp.float32
            ).astype(jnp.bfloat16)
            pa_ref[b] = partial[:, :DH2]
            pb_ref[b] = partial[:, DH2:]

        def sched(d):
            p1, p2, h0, kb = P1[d], P2[d], H0[d], KB[d]
            bp1, bp2, bh0, bkb = BP1[d], BP2[d], BH0[d], BKB[d]
            hb0, bhb0 = 2 - h0, 2 - bh0
            ki, bki = kb - h0, bkb - bh0
            sb, bsb = h0 + 1 - ki, bh0 + 1 - bki

            def mk(src, dst, sem, dev):
                return pltpu.make_async_remote_copy(
                    src_ref=src, dst_ref=dst,
                    send_sem=send_sems.at[sem], recv_sem=recv_sems.at[sem],
                    device_id=(dev,), device_id_type=pl.DeviceIdType.MESH,
                )

            a1 = mk(pa_ref.at[pl.ds(hb0, 2)], r1a_ref, 0, p1)
            b1 = mk(pb_ref.at[pl.ds(bhb0, 2)], r1b_ref, 4, bp1)
            if hb0 == bhb0:
                first = hb0
                order = [(first, None), (first + 1, "AB"),
                         (2 - first, None), (3 - first, None)]
            else:
                early = "A" if hb0 == 0 else "B"
                late = "B" if early == "A" else "A"
                order = [(0, None), (1, early), (2, None), (3, late)]
            for b, issue in order:
                compute_batch(b)
                if issue in ("A", "AB"):
                    a1.start()
                if issue in ("B", "AB"):
                    b1.start()
            a1.wait()
            b1.wait()
            aa_ref[...] = pa_ref[pl.ds(h0, 2)] + r1a_ref[...]
            ab_ref[...] = pb_ref[pl.ds(bh0, 2)] + r1b_ref[...]

            a2 = mk(aa_ref.at[1 - ki], ga_ref.at[sb], 1, p2)
            b2 = mk(ab_ref.at[1 - bki], gb_ref.at[bsb], 5, bp2)
            a2.start()
            b2.start()
            a2.wait()
            b2.wait()
            ga_ref[kb] = ga_ref[kb] + aa_ref[ki]
            gb_ref[bkb] = gb_ref[bkb] + ab_ref[bki]

            a3 = mk(ga_ref.at[kb], ga_ref.at[kb], 2, p2)
            b3 = mk(gb_ref.at[bkb], gb_ref.at[bkb], 6, bp2)
            a3.start()
            b3.start()
            out_ref[kb, :, :DH2] = ga_ref[kb].astype(jnp.float32)
            out_ref[bkb, :, DH2:] = gb_ref[bkb].astype(jnp.float32)
            a3.wait()
            b3.wait()

            a4 = mk(ga_ref.at[pl.ds(h0, 2)], ga_ref.at[pl.ds(h0, 2)], 3, p1)
            b4 = mk(gb_ref.at[pl.ds(bh0, 2)], gb_ref.at[pl.ds(bh0, 2)],
                    7, bp1)
            a4.start()
            b4.start()
            kb2, bkb2 = KB[p2], BKB[bp2]
            out_ref[kb2, :, :DH2] = ga_ref[kb2].astype(jnp.float32)
            out_ref[bkb2, :, DH2:] = gb_ref[bkb2].astype(jnp.float32)
            a4.wait()
            b4.wait()
            out_ref[pl.ds(hb0, 2), :, :DH2] = (
                ga_ref[pl.ds(hb0, 2)].astype(jnp.float32))
            out_ref[pl.ds(bhb0, 2), :, DH2:] = (
                gb_ref[pl.ds(bhb0, 2)].astype(jnp.float32))

        for d in range(N_DEV):
            @pl.when(my == d)
            def _(d=d):
                sched(d)

    return pl.pallas_call(
        body,
        out_shape=jax.ShapeDtypeStruct((B, SQ, D), jnp.float32),
        in_specs=[pl.BlockSpec(memory_space=pltpu.VMEM)] * 5,
        out_specs=pl.BlockSpec(memory_space=pltpu.VMEM),
        scratch_shapes=[
            pltpu.VMEM((B, SQ, DH2), jnp.bfloat16),
            pltpu.VMEM((B, SQ, DH2), jnp.bfloat16),
            pltpu.VMEM((2, SQ, DH2), jnp.bfloat16),
            pltpu.VMEM((2, SQ, DH2), jnp.bfloat16),
            pltpu.VMEM((2, SQ, DH2), jnp.bfloat16),
            pltpu.VMEM((2, SQ, DH2), jnp.bfloat16),
            pltpu.VMEM((B, SQ, DH2), jnp.bfloat16),
            pltpu.VMEM((B, SQ, DH2), jnp.bfloat16),
            pltpu.SemaphoreType.DMA((8,)),
            pltpu.SemaphoreType.DMA((8,)),
        ],
        compiler_params=pltpu.CompilerParams(collective_id=0),
    )(x, Wq, Wo, Wk, Wv)


# device time: 51156 ns/iter; 1.6501x vs baseline; 1.6501x over previous
import jax
import jax.numpy as jnp
from jax import lax
from jax.experimental import pallas as pl
from jax.experimental.pallas import tpu as pltpu

N_DEV = 4
B, SQ, D = 4, 256, 1024
DH2 = D // 2
HQ, DH = 8, 128
SCALE = 0.08838834764831843

P1 = [1, 0, 3, 2]
P2 = [3, 2, 1, 0]
H0 = [0, 2, 2, 0]
KB = [0, 2, 3, 1]
BP1 = [3, 2, 1, 0]
BP2 = [1, 0, 3, 2]
BH0 = [0, 0, 2, 2]
BKB = [0, 1, 2, 3]


def kernel(x, Wq, Wo, Wk, Wv):
    def body(x_ref, wq_ref, wo_ref, wk_ref, wv_ref, out_ref,
             pa_ref, pb_ref, r1a_ref, r1b_ref, aa_ref, ab_ref,
             ga_ref, gb_ref, send_sems, recv_sems):
        my = lax.axis_index("i")
        left = (my - 1) % N_DEV
        right = (my + 1) % N_DEV

        barrier_sem = pltpu.get_barrier_semaphore()
        for nbr in (left, right):
            pl.semaphore_signal(
                barrier_sem, inc=1,
                device_id=(nbr,), device_id_type=pl.DeviceIdType.MESH,
            )
        pl.semaphore_wait(barrier_sem, 2)

        wq = wq_ref[...].astype(jnp.bfloat16)
        wk = wk_ref[...].astype(jnp.bfloat16)
        wv = wv_ref[...].astype(jnp.bfloat16)
        wo = wo_ref[...].astype(jnp.bfloat16)

        x2 = x_ref[...].reshape(B * SQ, D).astype(jnp.bfloat16)
        q2 = jnp.dot(x2, wq, preferred_element_type=jnp.float32
                     ).astype(jnp.bfloat16)
        k2 = jnp.dot(x2, wk, preferred_element_type=jnp.float32
                     ).astype(jnp.bfloat16)
        v2 = jnp.dot(x2, wv, preferred_element_type=jnp.float32
                     ).astype(jnp.bfloat16)

        def compute_batch(b):
            rows = slice(b * SQ, (b + 1) * SQ)
            q3 = jnp.stack(
                [q2[rows, h * DH:(h + 1) * DH] for h in range(HQ)])
            k3 = jnp.stack(
                [k2[rows, h * DH:(h + 1) * DH] for h in range(HQ)])
            v3 = jnp.stack(
                [v2[rows, h * DH:(h + 1) * DH] for h in range(HQ)])
            s3 = lax.dot_general(
                q3, k3, (((2,), (2,)), ((0,), (0,))),
                preferred_element_type=jnp.float32,
            ) * SCALE
            m = jnp.max(s3, axis=2, keepdims=True)
            p = jnp.exp(s3 - m)
            l = jnp.sum(p, axis=2, keepdims=True)
            o3 = lax.dot_general(
                p.astype(jnp.bfloat16), v3, (((2,), (1,)), ((0,), (0,))),
                preferred_element_type=jnp.float32,
            ) / l
            ob = jnp.concatenate(
                [o3[h].astype(jnp.bfloat16) for h in range(HQ)], axis=1)
            partial = jnp.dot(
                ob, wo, preferred_element_type=jnp.float32
            ).astype(jnp.bfloat16)
            pa_ref[b] = partial[:, :DH2]
            pb_ref[b] = partial[:, DH2:]

        def mk(src, dst, sem, dev):
            return pltpu.make_async_remote_copy(
                src_ref=src, dst_ref=dst,
                send_sem=send_sems.at[sem], recv_sem=recv_sems.at[sem],
                device_id=(dev,), device_id_type=pl.DeviceIdType.MESH,
            )

        def sched(d):
            p1, p2, h0, kb = P1[d], P2[d], H0[d], KB[d]
            bp1, bp2, bh0, bkb = BP1[d], BP2[d], BH0[d], BKB[d]
            hb0, bhb0 = 2 - h0, 2 - bh0
            ki, bki = kb - h0, bkb - bh0
            sb, bsb = h0 + 1 - ki, bh0 + 1 - bki

            for slot in (0, 1):
                mk(pa_ref.at[hb0 + slot], r1a_ref.at[slot],
                   slot, p1).wait()
                mk(pb_ref.at[bhb0 + slot], r1b_ref.at[slot],
                   4 + slot, bp1).wait()
            aa_ref[...] = pa_ref[pl.ds(h0, 2)] + r1a_ref[...]
            ab_ref[...] = pb_ref[pl.ds(bh0, 2)] + r1b_ref[...]

            a2 = mk(aa_ref.at[1 - ki], ga_ref.at[sb], 2, p2)
            b2 = mk(ab_ref.at[1 - bki], gb_ref.at[bsb], 6, bp2)
            a2.start()
            b2.start()
            a2.wait()
            b2.wait()
            ga_ref[kb] = ga_ref[kb] + aa_ref[ki]
            gb_ref[bkb] = gb_ref[bkb] + ab_ref[bki]

            a3 = mk(ga_ref.at[kb], ga_ref.at[kb], 3, p2)
            b3 = mk(gb_ref.at[bkb], gb_ref.at[bkb], 7, bp2)
            a3.start()
            b3.start()
            out_ref[kb, :, :DH2] = ga_ref[kb]
            out_ref[bkb, :, DH2:] = gb_ref[bkb]
            a3.wait()
            b3.wait()

            a4 = mk(ga_ref.at[pl.ds(h0, 2)], ga_ref.at[pl.ds(h0, 2)], 8, p1)
            b4 = mk(gb_ref.at[pl.ds(bh0, 2)], gb_ref.at[pl.ds(bh0, 2)],
                    9, bp1)
            a4.start()
            b4.start()
            kb2, bkb2 = KB[p2], BKB[bp2]
            out_ref[kb2, :, :DH2] = ga_ref[kb2]
            out_ref[bkb2, :, DH2:] = gb_ref[bkb2]
            a4.wait()
            b4.wait()
            out_ref[pl.ds(hb0, 2), :, :DH2] = ga_ref[pl.ds(hb0, 2)]
            out_ref[pl.ds(bhb0, 2), :, DH2:] = gb_ref[pl.ds(bhb0, 2)]

        for b in range(B):
            compute_batch(b)
            slot = b % 2
            a_cond = ((my == 0) | (my == 3)) if b >= 2 else \
                     ((my == 1) | (my == 2))
            b_cond = ((my == 0) | (my == 1)) if b >= 2 else \
                     ((my == 2) | (my == 3))

            @pl.when(a_cond)
            def _(b=b, slot=slot):
                mk(pa_ref.at[b], r1a_ref.at[slot], slot, my ^ 1).start()

            @pl.when(b_cond)
            def _(b=b, slot=slot):
                mk(pb_ref.at[b], r1b_ref.at[slot], 4 + slot, 3 - my).start()

        for d in range(N_DEV):
            @pl.when(my == d)
            def _(d=d):
                sched(d)

    return pl.pallas_call(
        body,
        out_shape=jax.ShapeDtypeStruct((B, SQ, D), jnp.bfloat16),
        in_specs=[pl.BlockSpec(memory_space=pltpu.VMEM)] * 5,
        out_specs=pl.BlockSpec(memory_space=pltpu.VMEM),
        scratch_shapes=[
            pltpu.VMEM((B, SQ, DH2), jnp.bfloat16),
            pltpu.VMEM((B, SQ, DH2), jnp.bfloat16),
            pltpu.VMEM((2, SQ, DH2), jnp.bfloat16),
            pltpu.VMEM((2, SQ, DH2), jnp.bfloat16),
            pltpu.VMEM((2, SQ, DH2), jnp.bfloat16),
            pltpu.VMEM((2, SQ, DH2), jnp.bfloat16),
            pltpu.VMEM((B, SQ, DH2), jnp.bfloat16),
            pltpu.VMEM((B, SQ, DH2), jnp.bfloat16),
            pltpu.SemaphoreType.DMA((10,)),
            pltpu.SemaphoreType.DMA((10,)),
        ],
        compiler_params=pltpu.CompilerParams(collective_id=0),
    )(x, Wq, Wo, Wk, Wv)
